# baseline (speedup 1.0000x reference)
import jax
import jax.numpy as jnp
from jax import lax
from jax.experimental import pallas as pl
from jax.experimental.pallas import tpu as pltpu
from jax.experimental.pallas import tpu_sc as plsc

N = 10000
E = 320000
D = 128

NC = 2
NS = 16
L = 16

C = 128
NCHUNK = 2560
E_PAD = NCHUNK * C
CPT = NCHUNK // NS
CPW = NCHUNK // (NC * NS)
N_ACC = 10240
SD_PAD = 2 * N_ACC
RPT = N_ACC // NS
G8 = 8

_SC_PARAMS = pltpu.CompilerParams(needs_layout_passes=False)
_SC_MESH = dict(core_axis_name="c", subcore_axis_name="s")


_BLK = 400
_GRID = N // _BLK


def _tc1_body(x_ref, w_ref, a_ref, hab_ref, sd_ref):
    h = jnp.dot(x_ref[...], w_ref[...], preferred_element_type=jnp.float32)
    hab_ref[...] = jnp.stack([h[:, :128], h[:, 128:]], axis=0)
    sd_ref[...] = jnp.dot(h, a_ref[...], preferred_element_type=jnp.float32)


def _tc_proj1(x, W1, A1):
    return pl.pallas_call(
        _tc1_body,
        grid=(_GRID,),
        in_specs=[
            pl.BlockSpec((_BLK, 128), lambda i: (i, 0)),
            pl.BlockSpec((128, 256), lambda i: (0, 0)),
            pl.BlockSpec((256, 2), lambda i: (0, 0)),
        ],
        out_specs=[
            pl.BlockSpec((2, _BLK, 128), lambda i: (0, i, 0)),
            pl.BlockSpec((_BLK, 2), lambda i: (i, 0)),
        ],
        out_shape=[
            jax.ShapeDtypeStruct((2, N, 128), jnp.float32),
            jax.ShapeDtypeStruct((N, 2), jnp.float32),
        ],
    )(x, W1, A1)


def _tc2_body(oa_ref, ob_ref, b_ref, w_ref, a_ref, t2_ref, sd_ref):
    hcat = jnp.concatenate([oa_ref[...], ob_ref[...]], axis=1) + b_ref[...]
    z = jnp.where(hcat > 0, hcat, jnp.exp(hcat) - 1.0)
    h2 = jnp.dot(z, w_ref[...], preferred_element_type=jnp.float32)
    t2_ref[...] = jnp.concatenate(
        [h2, jnp.zeros((_BLK, 64), jnp.float32)], axis=1)
    sd_ref[...] = jnp.dot(h2, a_ref[...], preferred_element_type=jnp.float32)


def _tc_proj2(oa, ob, b1, W2, A2):
    return pl.pallas_call(
        _tc2_body,
        grid=(_GRID,),
        in_specs=[
            pl.BlockSpec((_BLK, 128), lambda i: (i, 0)),
            pl.BlockSpec((_BLK, 128), lambda i: (i, 0)),
            pl.BlockSpec((1, 256), lambda i: (0, 0)),
            pl.BlockSpec((256, 64), lambda i: (0, 0)),
            pl.BlockSpec((64, 2), lambda i: (0, 0)),
        ],
        out_specs=[
            pl.BlockSpec((_BLK, 128), lambda i: (i, 0)),
            pl.BlockSpec((_BLK, 2), lambda i: (i, 0)),
        ],
        out_shape=[
            jax.ShapeDtypeStruct((N, 128), jnp.float32),
            jax.ShapeDtypeStruct((N, 2), jnp.float32),
        ],
    )(oa, ob, b1, W2, A2)


def _tc3_body(p0_ref, p1_ref, d0_ref, d1_ref, b_ref, w_ref, a_ref,
              h3_ref, sd_ref):
    den = d0_ref[...] + d1_ref[...] + 1e-16
    h2 = (p0_ref[...] + p1_ref[...]) / den
    hcat = h2 + b_ref[...]
    z = jnp.where(hcat > 0, hcat, jnp.exp(hcat) - 1.0)
    h3 = jnp.dot(z, w_ref[...], preferred_element_type=jnp.float32)
    h3_ref[...] = h3
    sd_ref[...] = jnp.dot(h3, a_ref[...], preferred_element_type=jnp.float32)


def _tc_proj3(p0, p1, d0, d1, b2, W3, A3):
    return pl.pallas_call(
        _tc3_body,
        grid=(_GRID,),
        in_specs=[
            pl.BlockSpec((_BLK, 64), lambda i: (i, 0)),
            pl.BlockSpec((_BLK, 64), lambda i: (i, 0)),
            pl.BlockSpec((_BLK, 1), lambda i: (i, 0)),
            pl.BlockSpec((_BLK, 1), lambda i: (i, 0)),
            pl.BlockSpec((1, 64), lambda i: (0, 0)),
            pl.BlockSpec((64, 1), lambda i: (0, 0)),
            pl.BlockSpec((1, 2), lambda i: (0, 0)),
        ],
        out_specs=[
            pl.BlockSpec((_BLK, 1), lambda i: (i, 0)),
            pl.BlockSpec((_BLK, 2), lambda i: (i, 0)),
        ],
        out_shape=[
            jax.ShapeDtypeStruct((N, 1), jnp.float32),
            jax.ShapeDtypeStruct((N, 2), jnp.float32),
        ],
    )(p0, p1, d0, d1, b2, W3, A3)


def _zero_vec(buf, n):
    def zv(k, _):
        buf[pl.ds(k * L, L)] = jnp.zeros((L,), jnp.float32)
        return 0

    lax.fori_loop(0, n // L, zv, 0)


def _zero_rows(rowb, nrows, ncols):
    def zrow(r, _):
        for f in range(ncols // L):
            rowb[r, pl.ds(f * L, L)] = jnp.zeros((L,), jnp.float32)
        return 0

    lax.fori_loop(0, nrows, zrow, 0)


def _pha_body(src2d, dst2d, sd_hbm, ee2d, d0_hbm, d1_hbm,
              sdv, srcb, dstb, eec, zvec, dsh):
    c = lax.axis_index("c")
    s = lax.axis_index("s")
    cbase = (c * NS + s) * CPW

    pltpu.sync_copy(sd_hbm, sdv)
    pltpu.sync_copy(src2d.at[pl.ds(cbase, CPW)], srcb)
    pltpu.sync_copy(dst2d.at[pl.ds(cbase, CPW)], dstb)
    _zero_vec(zvec, RPT)
    pltpu.sync_copy(zvec, dsh.at[pl.ds(s * RPT, RPT)])
    plsc.subcore_barrier()

    def group(ii, _):
        def chunk(slot, _):
            i = ii * G8 + slot

            def vec(j, _):
                off = j * L
                vsrc = srcb[i, pl.ds(off, L)]
                vdst = dstb[i, pl.ds(off, L)]
                ss = plsc.load_gather(sdv, [vsrc * 2])
                dd = plsc.load_gather(sdv, [vdst * 2 + 1])
                e = ss + dd
                e = jnp.maximum(e, e * 0.2)
                eec[slot, pl.ds(off, L)] = jnp.exp(e)
                return 0

            lax.fori_loop(0, C // L, vec, 0)
            pltpu.sync_copy(eec.at[slot], dsh.at[dstb.at[i]], add=True)
            return 0

        lax.fori_loop(0, G8, chunk, 0)
        pltpu.sync_copy(eec, ee2d.at[pl.ds(cbase + ii * G8, G8)])
        return 0

    lax.fori_loop(0, CPW // G8, group, 0)
    plsc.subcore_barrier()

    @pl.when(c == 0)
    def _():
        pltpu.sync_copy(dsh.at[pl.ds(s * RPT, RPT)],
                        d0_hbm.at[pl.ds(s * RPT, RPT)])

    @pl.when(c == 1)
    def _():
        pltpu.sync_copy(dsh.at[pl.ds(s * RPT, RPT)],
                        d1_hbm.at[pl.ds(s * RPT, RPT)])


_sc_pha = pl.kernel(
    _pha_body,
    out_type=[
        jax.ShapeDtypeStruct((NCHUNK, C), jnp.float32),
        jax.ShapeDtypeStruct((N_ACC,), jnp.float32),
        jax.ShapeDtypeStruct((N_ACC,), jnp.float32),
    ],
    mesh=plsc.VectorSubcoreMesh(**_SC_MESH),
    compiler_params=_SC_PARAMS,
    scratch_types=[
        pltpu.VMEM((SD_PAD,), jnp.float32),
        pltpu.VMEM((CPW, C), jnp.int32),
        pltpu.VMEM((CPW, C), jnp.int32),
        pltpu.VMEM((G8, C), jnp.float32),
        pltpu.VMEM((RPT,), jnp.float32),
        pltpu.VMEM_SHARED((N_ACC,), jnp.float32),
    ],
)


def _agg_chunks(src_sl, dst2d, ee2d, tab_hbm, cbase, nchunks,
                idx3, dst3, ee3, rowb2, acc, gsem0, gsem1, psem,
                nscale=128 // L):
    ngroups = nchunks // G8

    def meta_issue(ii):
        row0 = cbase + ii * G8
        b = lax.rem(ii, 2)
        pltpu.async_copy(src_sl(row0), idx3.at[b], psem)
        pltpu.async_copy(dst2d.at[pl.ds(row0, G8)], dst3.at[b], psem)
        pltpu.async_copy(ee2d.at[pl.ds(row0, G8)], ee3.at[b], psem)

    def meta_drain(ii):
        row0 = cbase + ii * G8
        b = lax.rem(ii, 2)
        pltpu.make_async_copy(src_sl(row0), idx3.at[b], psem).wait()
        pltpu.make_async_copy(dst2d.at[pl.ds(row0, G8)], dst3.at[b],
                              psem).wait()
        pltpu.make_async_copy(ee2d.at[pl.ds(row0, G8)], ee3.at[b],
                              psem).wait()

    def gather_issue(g, sem):
        ii = g // G8
        slot = lax.rem(g, G8)
        gb = lax.rem(g, 2)
        pltpu.async_copy(tab_hbm.at[idx3.at[lax.rem(ii, 2), slot]],
                         rowb2.at[gb], sem)

    def gather_wait(g, sem):
        gb = lax.rem(g, 2)
        pltpu.make_async_copy(tab_hbm.at[idx3.at[0, 0]],
                              rowb2.at[gb], sem).wait()

    meta_issue(0)
    meta_drain(0)
    gather_issue(0, gsem0)
    gather_issue(1, gsem1)

    def group(ii, _):
        @pl.when(ii + 1 < ngroups)
        def _():
            meta_issue(ii + 1)

        ib = lax.rem(ii, 2)

        def chunk(slot, _):
            g = ii * G8 + slot
            gb = lax.rem(g, 2)

            @pl.when(gb == 0)
            def _():
                gather_wait(g, gsem0)

            @pl.when(gb == 1)
            def _():
                gather_wait(g, gsem1)

            def sgroup(grp, _):
                av = ee3[ib, slot, pl.ds(grp * L, L)]
                for r in range(L):
                    a = av[r]
                    row = grp * L + r
                    for f in range(nscale):
                        rowb2[gb, row, pl.ds(f * L, L)] = (
                            rowb2[gb, row, pl.ds(f * L, L)] * a)
                return 0

            lax.fori_loop(0, C // L, sgroup, 0)
            pltpu.sync_copy(rowb2.at[gb], acc.at[dst3.at[ib, slot]], add=True)

            @pl.when(slot == G8 - 3)
            def _():
                @pl.when(ii + 1 < ngroups)
                def _():
                    meta_drain(ii + 1)

            @pl.when(g + 2 < nchunks)
            def _():
                @pl.when(gb == 0)
                def _():
                    gather_issue(g + 2, gsem0)

                @pl.when(gb == 1)
                def _():
                    gather_issue(g + 2, gsem1)

            return 0

        lax.fori_loop(0, G8, chunk, 0)
        return 0

    lax.fori_loop(0, ngroups, group, 0)


def _phb1_body(src3, dst2d, ee2d, d0_hbm, d1_hbm, hab, oa, ob,
               idx3, dst3, ee3, rowb2, dvec, dvec2, acc,
               gsem0, gsem1, psem):
    c = lax.axis_index("c")
    s = lax.axis_index("s")
    rbase = s * RPT

    _zero_rows(rowb2.at[0], C, 128)
    for k in range(RPT // C):
        pltpu.sync_copy(rowb2.at[0], acc.at[pl.ds(rbase + k * C, C)])
    plsc.subcore_barrier()

    _agg_chunks(lambda row0: src3.at[c, pl.ds(row0, G8)],
                dst2d, ee2d, hab, s * CPT, CPT,
                idx3, dst3, ee3, rowb2, acc, gsem0, gsem1, psem)

    plsc.subcore_barrier()

    def wchunk(k, _):
        r0 = rbase + k * C
        pltpu.sync_copy(acc.at[pl.ds(r0, C)], rowb2.at[0])
        pltpu.sync_copy(d0_hbm.at[pl.ds(r0, C)], dvec)
        pltpu.sync_copy(d1_hbm.at[pl.ds(r0, C)], dvec2)

        def sgroup(g, _):
            off = g * L
            rec = 1.0 / (dvec[pl.ds(off, L)] + dvec2[pl.ds(off, L)] + 1e-16)
            for r in range(L):
                a = rec[r]
                row = off + r
                for f in range(128 // L):
                    rowb2[0, row, pl.ds(f * L, L)] = (
                        rowb2[0, row, pl.ds(f * L, L)] * a)
            return 0

        lax.fori_loop(0, C // L, sgroup, 0)

        @pl.when(c == 0)
        def _():
            pltpu.sync_copy(rowb2.at[0], oa.at[pl.ds(r0, C)])

        @pl.when(c == 1)
        def _():
            pltpu.sync_copy(rowb2.at[0], ob.at[pl.ds(r0, C)])

        return 0

    lax.fori_loop(0, RPT // C, wchunk, 0)


_sc_phb1 = pl.kernel(
    _phb1_body,
    out_type=[
        jax.ShapeDtypeStruct((N_ACC, 128), jnp.float32),
        jax.ShapeDtypeStruct((N_ACC, 128), jnp.float32),
    ],
    mesh=plsc.VectorSubcoreMesh(**_SC_MESH),
    compiler_params=_SC_PARAMS,
    scratch_types=[
        pltpu.VMEM((2, G8, C), jnp.int32),
        pltpu.VMEM((2, G8, C), jnp.int32),
        pltpu.VMEM((2, G8, C), jnp.float32),
        pltpu.VMEM((2, C, 128), jnp.float32),
        pltpu.VMEM((C,), jnp.float32),
        pltpu.VMEM((C,), jnp.float32),
        pltpu.VMEM_SHARED((N_ACC, 128), jnp.float32),
        pltpu.SemaphoreType.DMA,
        pltpu.SemaphoreType.DMA,
        pltpu.SemaphoreType.DMA,
    ],
)


def _phb2_body(src2d, dst2d, ee2d, t2, p0, p1,
               idx3, dst3, ee3, rowb2, acc, gsem0, gsem1, psem):
    c = lax.axis_index("c")
    s = lax.axis_index("s")
    rbase = s * RPT
    cbase = (c * NS + s) * CPW

    _zero_rows(rowb2.at[0], C, 128)
    for k in range(RPT // C):
        pltpu.sync_copy(rowb2.at[0], acc.at[pl.ds(rbase + k * C, C)])
    plsc.subcore_barrier()

    _agg_chunks(lambda row0: src2d.at[pl.ds(row0, G8)],
                dst2d, ee2d, t2, cbase, CPW,
                idx3, dst3, ee3, rowb2, acc, gsem0, gsem1, psem,
                nscale=64 // L)
    plsc.subcore_barrier()

    def wchunk(k, _):
        r0 = rbase + k * C
        pltpu.sync_copy(acc.at[pl.ds(r0, C)], rowb2.at[0])

        @pl.when(c == 0)
        def _():
            pltpu.sync_copy(rowb2.at[0], p0.at[pl.ds(r0, C)])

        @pl.when(c == 1)
        def _():
            pltpu.sync_copy(rowb2.at[0], p1.at[pl.ds(r0, C)])

        return 0

    lax.fori_loop(0, RPT // C, wchunk, 0)


_sc_phb2 = pl.kernel(
    _phb2_body,
    out_type=[
        jax.ShapeDtypeStruct((N_ACC, 128), jnp.float32),
        jax.ShapeDtypeStruct((N_ACC, 128), jnp.float32),
    ],
    mesh=plsc.VectorSubcoreMesh(**_SC_MESH),
    compiler_params=_SC_PARAMS,
    scratch_types=[
        pltpu.VMEM((2, G8, C), jnp.int32),
        pltpu.VMEM((2, G8, C), jnp.int32),
        pltpu.VMEM((2, G8, C), jnp.float32),
        pltpu.VMEM((2, C, 128), jnp.float32),
        pltpu.VMEM_SHARED((N_ACC, 128), jnp.float32),
        pltpu.SemaphoreType.DMA,
        pltpu.SemaphoreType.DMA,
        pltpu.SemaphoreType.DMA,
    ],
)


def _sc3_body(src2d, dst2d, sd_hbm, h3_hbm, out_hbm,
              sdv, h3v, srcb, dstb, eec, zvec, dvec, osh, dsh):
    c = lax.axis_index("c")
    s = lax.axis_index("s")
    cbase = (c * NS + s) * CPW
    rbase = s * RPT

    pltpu.sync_copy(sd_hbm, sdv)
    pltpu.sync_copy(h3_hbm, h3v)
    pltpu.sync_copy(src2d.at[pl.ds(cbase, CPW)], srcb)
    pltpu.sync_copy(dst2d.at[pl.ds(cbase, CPW)], dstb)
    _zero_vec(zvec, RPT)
    pltpu.sync_copy(zvec, dsh.at[pl.ds(rbase, RPT)])
    pltpu.sync_copy(zvec, osh.at[pl.ds(rbase, RPT)])
    plsc.subcore_barrier()

    def chunk(i, _):
        def vec(j, _):
            off = j * L
            vsrc = srcb[i, pl.ds(off, L)]
            vdst = dstb[i, pl.ds(off, L)]
            ss = plsc.load_gather(sdv, [vsrc * 2])
            dd = plsc.load_gather(sdv, [vdst * 2 + 1])
            e = ss + dd
            e = jnp.maximum(e, e * 0.2)
            eec[0, pl.ds(off, L)] = jnp.exp(e)
            return 0

        lax.fori_loop(0, C // L, vec, 0)
        pltpu.sync_copy(eec.at[0], dsh.at[dstb.at[i]], add=True)

        def vec2(j, _):
            off = j * L
            vsrc = srcb[i, pl.ds(off, L)]
            vh = plsc.load_gather(h3v, [vsrc])
            eec[0, pl.ds(off, L)] = eec[0, pl.ds(off, L)] * vh
            return 0

        lax.fori_loop(0, C // L, vec2, 0)
        pltpu.sync_copy(eec.at[0], osh.at[dstb.at[i]], add=True)
        return 0

    lax.fori_loop(0, CPW, chunk, 0)
    plsc.subcore_barrier()

    @pl.when(c == 0)
    def _():
        pltpu.sync_copy(osh.at[pl.ds(rbase, RPT)], zvec)
        pltpu.sync_copy(dsh.at[pl.ds(rbase, RPT)], dvec)
        pltpu.sync_copy(zvec, out_hbm.at[pl.ds(rbase, RPT)])
        pltpu.sync_copy(dvec, out_hbm.at[pl.ds(2 * N_ACC + rbase, RPT)])

    @pl.when(c == 1)
    def _():
        pltpu.sync_copy(osh.at[pl.ds(rbase, RPT)], zvec)
        pltpu.sync_copy(dsh.at[pl.ds(rbase, RPT)], dvec)
        pltpu.sync_copy(zvec, out_hbm.at[pl.ds(N_ACC + rbase, RPT)])
        pltpu.sync_copy(dvec, out_hbm.at[pl.ds(3 * N_ACC + rbase, RPT)])


_sc_layer3 = pl.kernel(
    _sc3_body,
    out_type=jax.ShapeDtypeStruct((4 * N_ACC,), jnp.float32),
    mesh=plsc.VectorSubcoreMesh(**_SC_MESH),
    compiler_params=_SC_PARAMS,
    scratch_types=[
        pltpu.VMEM((SD_PAD,), jnp.float32),
        pltpu.VMEM((N,), jnp.float32),
        pltpu.VMEM((CPW, C), jnp.int32),
        pltpu.VMEM((CPW, C), jnp.int32),
        pltpu.VMEM((1, C), jnp.float32),
        pltpu.VMEM((RPT,), jnp.float32),
        pltpu.VMEM((RPT,), jnp.float32),
        pltpu.VMEM_SHARED((N_ACC,), jnp.float32),
        pltpu.VMEM_SHARED((N_ACC,), jnp.float32),
    ],
)


def _tcf_body(parts_ref, b_ref, out_ref):
    p = parts_ref[...]
    num = p[:, 0:1] + p[:, 1:2]
    den = p[:, 2:3] + p[:, 3:4] + 1e-16
    out_ref[...] = num / den + b_ref[...]


def _tc_final(parts, b3):
    return pl.pallas_call(
        _tcf_body,
        grid=(_GRID,),
        in_specs=[
            pl.BlockSpec((_BLK, 4), lambda i: (i, 0)),
            pl.BlockSpec((1, 1), lambda i: (0, 0)),
        ],
        out_specs=pl.BlockSpec((_BLK, 1), lambda i: (i, 0)),
        out_shape=jax.ShapeDtypeStruct((N, 1), jnp.float32),
    )(parts, b3)


def _flatten_sd(sd):
    flat = sd.reshape(-1)
    return jnp.concatenate([flat, jnp.zeros((SD_PAD - 2 * N,), jnp.float32)])


def kernel(x, edge_index, W1, a1s, a1d, b1, W2, a2s, a2d, b2, W3, a3s, a3d, b3):
    src = edge_index[0]
    dst = edge_index[1]

    pad = E_PAD - E
    apad = jnp.arange(pad, dtype=jnp.int32)
    src2d = jnp.concatenate([src, (apad * 131) % N]).reshape(NCHUNK, C)
    dst_p = jnp.concatenate([dst, N + apad % (N_ACC - N)])
    dst2d = dst_p.reshape(NCHUNK, C)

    A1 = jnp.stack([a1s, a1d], axis=1)
    A2 = jnp.stack([a2s, a2d], axis=1)
    A3 = jnp.stack([a3s, a3d], axis=1)

    hab, sd1 = _tc_proj1(x, W1, A1)
    src3 = jnp.stack([src2d, src2d + N])
    ee1, d10, d11 = _sc_pha(src2d, dst2d, _flatten_sd(sd1))
    oa1, ob1 = _sc_phb1(src3, dst2d, ee1, d10, d11, hab.reshape(2 * N, 128))

    t2, sd2 = _tc_proj2(oa1[:N], ob1[:N], b1[None, :], W2, A2)
    ee2, d20, d21 = _sc_pha(src2d, dst2d, _flatten_sd(sd2))
    p0, p1 = _sc_phb2(src2d, dst2d, ee2, t2)

    h3, sd3 = _tc_proj3(p0[:N, :64], p1[:N, :64],
                        d20[:N].reshape(N, 1), d21[:N].reshape(N, 1),
                        b2[None, :], W3, A3)
    parts3 = _sc_layer3(src2d, dst2d, _flatten_sd(sd3), h3.reshape(-1))
    parts3 = parts3.reshape(4, N_ACC)[:, :N].T

    return _tc_final(parts3, b3.reshape(1, 1))


# speedup vs baseline: 1.0001x; 1.0001x over previous
import jax
import jax.numpy as jnp
from jax import lax
from jax.experimental import pallas as pl
from jax.experimental.pallas import tpu as pltpu
from jax.experimental.pallas import tpu_sc as plsc

N = 10000
E = 320000
D = 128

NC = 2
NS = 16
L = 16

C = 128
NCHUNK = 2560
E_PAD = NCHUNK * C
CPT = NCHUNK // NS
CPW = NCHUNK // (NC * NS)
N_ACC = 10240
SD_PAD = 2 * N_ACC
RPT = N_ACC // NS
G8 = 8

_SC_PARAMS = pltpu.CompilerParams(needs_layout_passes=False)
_SC_MESH = dict(core_axis_name="c", subcore_axis_name="s")


_BLK = 400
_GRID = N // _BLK


def _tc1_body(x_ref, w_ref, a_ref, hab_ref, sd_ref):
    h = jnp.dot(x_ref[...], w_ref[...], preferred_element_type=jnp.float32)
    hab_ref[...] = jnp.stack([h[:, :128], h[:, 128:]], axis=0)
    sd_ref[...] = jnp.dot(h, a_ref[...], preferred_element_type=jnp.float32)


def _tc_proj1(x, W1, A1):
    return pl.pallas_call(
        _tc1_body,
        grid=(_GRID,),
        in_specs=[
            pl.BlockSpec((_BLK, 128), lambda i: (i, 0)),
            pl.BlockSpec((128, 256), lambda i: (0, 0)),
            pl.BlockSpec((256, 2), lambda i: (0, 0)),
        ],
        out_specs=[
            pl.BlockSpec((2, _BLK, 128), lambda i: (0, i, 0)),
            pl.BlockSpec((_BLK, 2), lambda i: (i, 0)),
        ],
        out_shape=[
            jax.ShapeDtypeStruct((2, N, 128), jnp.float32),
            jax.ShapeDtypeStruct((N, 2), jnp.float32),
        ],
    )(x, W1, A1)


def _tc2_body(oa_ref, ob_ref, b_ref, w_ref, a_ref, t2_ref, sd_ref):
    hcat = jnp.concatenate([oa_ref[...], ob_ref[...]], axis=1) + b_ref[...]
    z = jnp.where(hcat > 0, hcat, jnp.exp(hcat) - 1.0)
    h2 = jnp.dot(z, w_ref[...], preferred_element_type=jnp.float32)
    t2_ref[...] = jnp.concatenate(
        [h2, jnp.zeros((_BLK, 64), jnp.float32)], axis=1)
    sd_ref[...] = jnp.dot(h2, a_ref[...], preferred_element_type=jnp.float32)


def _tc_proj2(oa, ob, b1, W2, A2):
    return pl.pallas_call(
        _tc2_body,
        grid=(_GRID,),
        in_specs=[
            pl.BlockSpec((_BLK, 128), lambda i: (i, 0)),
            pl.BlockSpec((_BLK, 128), lambda i: (i, 0)),
            pl.BlockSpec((1, 256), lambda i: (0, 0)),
            pl.BlockSpec((256, 64), lambda i: (0, 0)),
            pl.BlockSpec((64, 2), lambda i: (0, 0)),
        ],
        out_specs=[
            pl.BlockSpec((_BLK, 128), lambda i: (i, 0)),
            pl.BlockSpec((_BLK, 2), lambda i: (i, 0)),
        ],
        out_shape=[
            jax.ShapeDtypeStruct((N, 128), jnp.float32),
            jax.ShapeDtypeStruct((N, 2), jnp.float32),
        ],
    )(oa, ob, b1, W2, A2)


def _tc3_body(p0_ref, p1_ref, d0_ref, d1_ref, b_ref, w_ref, a_ref,
              h3_ref, sd_ref):
    den = d0_ref[...] + d1_ref[...] + 1e-16
    h2 = (p0_ref[...] + p1_ref[...]) / den
    hcat = h2 + b_ref[...]
    z = jnp.where(hcat > 0, hcat, jnp.exp(hcat) - 1.0)
    h3 = jnp.dot(z, w_ref[...], preferred_element_type=jnp.float32)
    h3_ref[...] = h3
    sd_ref[...] = jnp.dot(h3, a_ref[...], preferred_element_type=jnp.float32)


def _tc_proj3(p0, p1, d0, d1, b2, W3, A3):
    return pl.pallas_call(
        _tc3_body,
        grid=(_GRID,),
        in_specs=[
            pl.BlockSpec((_BLK, 64), lambda i: (i, 0)),
            pl.BlockSpec((_BLK, 64), lambda i: (i, 0)),
            pl.BlockSpec((_BLK, 1), lambda i: (i, 0)),
            pl.BlockSpec((_BLK, 1), lambda i: (i, 0)),
            pl.BlockSpec((1, 64), lambda i: (0, 0)),
            pl.BlockSpec((64, 1), lambda i: (0, 0)),
            pl.BlockSpec((1, 2), lambda i: (0, 0)),
        ],
        out_specs=[
            pl.BlockSpec((_BLK, 1), lambda i: (i, 0)),
            pl.BlockSpec((_BLK, 2), lambda i: (i, 0)),
        ],
        out_shape=[
            jax.ShapeDtypeStruct((N, 1), jnp.float32),
            jax.ShapeDtypeStruct((N, 2), jnp.float32),
        ],
    )(p0, p1, d0, d1, b2, W3, A3)


def _zero_vec(buf, n):
    def zv(k, _):
        buf[pl.ds(k * L, L)] = jnp.zeros((L,), jnp.float32)
        return 0

    lax.fori_loop(0, n // L, zv, 0)


def _zero_rows(rowb, nrows, ncols):
    def zrow(r, _):
        for f in range(ncols // L):
            rowb[r, pl.ds(f * L, L)] = jnp.zeros((L,), jnp.float32)
        return 0

    lax.fori_loop(0, nrows, zrow, 0)


def _pha_body(src2d, dst2d, sd_hbm, ee2d, d0_hbm, d1_hbm,
              sdv, srcb, dstb, eec, zvec, dsh):
    c = lax.axis_index("c")
    s = lax.axis_index("s")
    cbase = (c * NS + s) * CPW

    pltpu.sync_copy(sd_hbm, sdv)
    pltpu.sync_copy(src2d.at[pl.ds(cbase, CPW)], srcb)
    pltpu.sync_copy(dst2d.at[pl.ds(cbase, CPW)], dstb)
    _zero_vec(zvec, RPT)
    pltpu.sync_copy(zvec, dsh.at[pl.ds(s * RPT, RPT)])
    plsc.subcore_barrier()

    def group(ii, _):
        def chunk(slot, _):
            i = ii * G8 + slot

            def vec(j, _):
                off = j * L
                vsrc = srcb[i, pl.ds(off, L)]
                vdst = dstb[i, pl.ds(off, L)]
                ss = plsc.load_gather(sdv, [vsrc * 2])
                dd = plsc.load_gather(sdv, [vdst * 2 + 1])
                e = ss + dd
                e = jnp.maximum(e, e * 0.2)
                eec[slot, pl.ds(off, L)] = jnp.exp(e)
                return 0

            lax.fori_loop(0, C // L, vec, 0)
            pltpu.sync_copy(eec.at[slot], dsh.at[dstb.at[i]], add=True)
            return 0

        lax.fori_loop(0, G8, chunk, 0)
        pltpu.sync_copy(eec, ee2d.at[pl.ds(cbase + ii * G8, G8)])
        return 0

    lax.fori_loop(0, CPW // G8, group, 0)
    plsc.subcore_barrier()

    @pl.when(c == 0)
    def _():
        pltpu.sync_copy(dsh.at[pl.ds(s * RPT, RPT)],
                        d0_hbm.at[pl.ds(s * RPT, RPT)])

    @pl.when(c == 1)
    def _():
        pltpu.sync_copy(dsh.at[pl.ds(s * RPT, RPT)],
                        d1_hbm.at[pl.ds(s * RPT, RPT)])


_sc_pha = pl.kernel(
    _pha_body,
    out_type=[
        jax.ShapeDtypeStruct((NCHUNK, C), jnp.float32),
        jax.ShapeDtypeStruct((N_ACC,), jnp.float32),
        jax.ShapeDtypeStruct((N_ACC,), jnp.float32),
    ],
    mesh=plsc.VectorSubcoreMesh(**_SC_MESH),
    compiler_params=_SC_PARAMS,
    scratch_types=[
        pltpu.VMEM((SD_PAD,), jnp.float32),
        pltpu.VMEM((CPW, C), jnp.int32),
        pltpu.VMEM((CPW, C), jnp.int32),
        pltpu.VMEM((G8, C), jnp.float32),
        pltpu.VMEM((RPT,), jnp.float32),
        pltpu.VMEM_SHARED((N_ACC,), jnp.float32),
    ],
)


def _agg_chunks(src_sl, dst2d, ee2d, tab_hbm, cbase, nchunks,
                idx3, dst3, ee3, rowb2, acc, gsem0, gsem1, psem,
                nscale=128 // L, goff=0):
    ngroups = nchunks // G8

    def row_of(ii):
        return cbase + lax.rem(ii + goff, ngroups) * G8

    def meta_issue(ii):
        row0 = row_of(ii)
        b = lax.rem(ii, 2)
        pltpu.async_copy(src_sl(row0), idx3.at[b], psem)
        pltpu.async_copy(dst2d.at[pl.ds(row0, G8)], dst3.at[b], psem)
        pltpu.async_copy(ee2d.at[pl.ds(row0, G8)], ee3.at[b], psem)

    def meta_drain(ii):
        row0 = row_of(ii)
        b = lax.rem(ii, 2)
        pltpu.make_async_copy(src_sl(row0), idx3.at[b], psem).wait()
        pltpu.make_async_copy(dst2d.at[pl.ds(row0, G8)], dst3.at[b],
                              psem).wait()
        pltpu.make_async_copy(ee2d.at[pl.ds(row0, G8)], ee3.at[b],
                              psem).wait()

    def gather_issue(g, sem):
        ii = g // G8
        slot = lax.rem(g, G8)
        gb = lax.rem(g, 2)
        pltpu.async_copy(tab_hbm.at[idx3.at[lax.rem(ii, 2), slot]],
                         rowb2.at[gb], sem)

    def gather_wait(g, sem):
        gb = lax.rem(g, 2)
        pltpu.make_async_copy(tab_hbm.at[idx3.at[0, 0]],
                              rowb2.at[gb], sem).wait()

    meta_issue(0)
    meta_drain(0)
    gather_issue(0, gsem0)
    gather_issue(1, gsem1)

    def group(ii, _):
        @pl.when(ii + 1 < ngroups)
        def _():
            meta_issue(ii + 1)

        ib = lax.rem(ii, 2)

        def chunk(slot, _):
            g = ii * G8 + slot
            gb = lax.rem(g, 2)

            @pl.when(gb == 0)
            def _():
                gather_wait(g, gsem0)

            @pl.when(gb == 1)
            def _():
                gather_wait(g, gsem1)

            def sgroup(grp, _):
                av = ee3[ib, slot, pl.ds(grp * L, L)]
                for r in range(L):
                    a = av[r]
                    row = grp * L + r
                    for f in range(nscale):
                        rowb2[gb, row, pl.ds(f * L, L)] = (
                            rowb2[gb, row, pl.ds(f * L, L)] * a)
                return 0

            lax.fori_loop(0, C // L, sgroup, 0)
            pltpu.sync_copy(rowb2.at[gb], acc.at[dst3.at[ib, slot]], add=True)

            @pl.when(slot == G8 - 3)
            def _():
                @pl.when(ii + 1 < ngroups)
                def _():
                    meta_drain(ii + 1)

            @pl.when(g + 2 < nchunks)
            def _():
                @pl.when(gb == 0)
                def _():
                    gather_issue(g + 2, gsem0)

                @pl.when(gb == 1)
                def _():
                    gather_issue(g + 2, gsem1)

            return 0

        lax.fori_loop(0, G8, chunk, 0)
        return 0

    lax.fori_loop(0, ngroups, group, 0)


def _phb1_body(src3, dst2d, ee2d, d0_hbm, d1_hbm, hab, oa, ob,
               idx3, dst3, ee3, rowb2, dvec, dvec2, acc,
               gsem0, gsem1, psem):
    c = lax.axis_index("c")
    s = lax.axis_index("s")
    rbase = s * RPT

    _zero_rows(rowb2.at[0], C, 128)
    for k in range(RPT // C):
        pltpu.sync_copy(rowb2.at[0], acc.at[pl.ds(rbase + k * C, C)])
    plsc.subcore_barrier()

    _agg_chunks(lambda row0: src3.at[c, pl.ds(row0, G8)],
                dst2d, ee2d, hab, s * CPT, CPT,
                idx3, dst3, ee3, rowb2, acc, gsem0, gsem1, psem,
                goff=c * ((CPT // G8) // 2))

    plsc.subcore_barrier()

    def wchunk(k, _):
        r0 = rbase + k * C
        pltpu.sync_copy(acc.at[pl.ds(r0, C)], rowb2.at[0])
        pltpu.sync_copy(d0_hbm.at[pl.ds(r0, C)], dvec)
        pltpu.sync_copy(d1_hbm.at[pl.ds(r0, C)], dvec2)

        def sgroup(g, _):
            off = g * L
            rec = 1.0 / (dvec[pl.ds(off, L)] + dvec2[pl.ds(off, L)] + 1e-16)
            for r in range(L):
                a = rec[r]
                row = off + r
                for f in range(128 // L):
                    rowb2[0, row, pl.ds(f * L, L)] = (
                        rowb2[0, row, pl.ds(f * L, L)] * a)
            return 0

        lax.fori_loop(0, C // L, sgroup, 0)

        @pl.when(c == 0)
        def _():
            pltpu.sync_copy(rowb2.at[0], oa.at[pl.ds(r0, C)])

        @pl.when(c == 1)
        def _():
            pltpu.sync_copy(rowb2.at[0], ob.at[pl.ds(r0, C)])

        return 0

    lax.fori_loop(0, RPT // C, wchunk, 0)


_sc_phb1 = pl.kernel(
    _phb1_body,
    out_type=[
        jax.ShapeDtypeStruct((N_ACC, 128), jnp.float32),
        jax.ShapeDtypeStruct((N_ACC, 128), jnp.float32),
    ],
    mesh=plsc.VectorSubcoreMesh(**_SC_MESH),
    compiler_params=_SC_PARAMS,
    scratch_types=[
        pltpu.VMEM((2, G8, C), jnp.int32),
        pltpu.VMEM((2, G8, C), jnp.int32),
        pltpu.VMEM((2, G8, C), jnp.float32),
        pltpu.VMEM((2, C, 128), jnp.float32),
        pltpu.VMEM((C,), jnp.float32),
        pltpu.VMEM((C,), jnp.float32),
        pltpu.VMEM_SHARED((N_ACC, 128), jnp.float32),
        pltpu.SemaphoreType.DMA,
        pltpu.SemaphoreType.DMA,
        pltpu.SemaphoreType.DMA,
    ],
)


def _phb2_body(src2d, dst2d, ee2d, t2, p0, p1,
               idx3, dst3, ee3, rowb2, acc, gsem0, gsem1, psem):
    c = lax.axis_index("c")
    s = lax.axis_index("s")
    rbase = s * RPT
    cbase = (c * NS + s) * CPW

    _zero_rows(rowb2.at[0], C, 128)
    for k in range(RPT // C):
        pltpu.sync_copy(rowb2.at[0], acc.at[pl.ds(rbase + k * C, C)])
    plsc.subcore_barrier()

    _agg_chunks(lambda row0: src2d.at[pl.ds(row0, G8)],
                dst2d, ee2d, t2, cbase, CPW,
                idx3, dst3, ee3, rowb2, acc, gsem0, gsem1, psem,
                nscale=64 // L)
    plsc.subcore_barrier()

    def wchunk(k, _):
        r0 = rbase + k * C
        pltpu.sync_copy(acc.at[pl.ds(r0, C)], rowb2.at[0])

        @pl.when(c == 0)
        def _():
            pltpu.sync_copy(rowb2.at[0], p0.at[pl.ds(r0, C)])

        @pl.when(c == 1)
        def _():
            pltpu.sync_copy(rowb2.at[0], p1.at[pl.ds(r0, C)])

        return 0

    lax.fori_loop(0, RPT // C, wchunk, 0)


_sc_phb2 = pl.kernel(
    _phb2_body,
    out_type=[
        jax.ShapeDtypeStruct((N_ACC, 128), jnp.float32),
        jax.ShapeDtypeStruct((N_ACC, 128), jnp.float32),
    ],
    mesh=plsc.VectorSubcoreMesh(**_SC_MESH),
    compiler_params=_SC_PARAMS,
    scratch_types=[
        pltpu.VMEM((2, G8, C), jnp.int32),
        pltpu.VMEM((2, G8, C), jnp.int32),
        pltpu.VMEM((2, G8, C), jnp.float32),
        pltpu.VMEM((2, C, 128), jnp.float32),
        pltpu.VMEM_SHARED((N_ACC, 128), jnp.float32),
        pltpu.SemaphoreType.DMA,
        pltpu.SemaphoreType.DMA,
        pltpu.SemaphoreType.DMA,
    ],
)


def _sc3_body(src2d, dst2d, sd_hbm, h3_hbm, out_hbm,
              sdv, h3v, srcb, dstb, eec, zvec, dvec, osh, dsh):
    c = lax.axis_index("c")
    s = lax.axis_index("s")
    cbase = (c * NS + s) * CPW
    rbase = s * RPT

    pltpu.sync_copy(sd_hbm, sdv)
    pltpu.sync_copy(h3_hbm, h3v)
    pltpu.sync_copy(src2d.at[pl.ds(cbase, CPW)], srcb)
    pltpu.sync_copy(dst2d.at[pl.ds(cbase, CPW)], dstb)
    _zero_vec(zvec, RPT)
    pltpu.sync_copy(zvec, dsh.at[pl.ds(rbase, RPT)])
    pltpu.sync_copy(zvec, osh.at[pl.ds(rbase, RPT)])
    plsc.subcore_barrier()

    def chunk(i, _):
        def vec(j, _):
            off = j * L
            vsrc = srcb[i, pl.ds(off, L)]
            vdst = dstb[i, pl.ds(off, L)]
            ss = plsc.load_gather(sdv, [vsrc * 2])
            dd = plsc.load_gather(sdv, [vdst * 2 + 1])
            e = ss + dd
            e = jnp.maximum(e, e * 0.2)
            eec[0, pl.ds(off, L)] = jnp.exp(e)
            return 0

        lax.fori_loop(0, C // L, vec, 0)
        pltpu.sync_copy(eec.at[0], dsh.at[dstb.at[i]], add=True)

        def vec2(j, _):
            off = j * L
            vsrc = srcb[i, pl.ds(off, L)]
            vh = plsc.load_gather(h3v, [vsrc])
            eec[0, pl.ds(off, L)] = eec[0, pl.ds(off, L)] * vh
            return 0

        lax.fori_loop(0, C // L, vec2, 0)
        pltpu.sync_copy(eec.at[0], osh.at[dstb.at[i]], add=True)
        return 0

    lax.fori_loop(0, CPW, chunk, 0)
    plsc.subcore_barrier()

    @pl.when(c == 0)
    def _():
        pltpu.sync_copy(osh.at[pl.ds(rbase, RPT)], zvec)
        pltpu.sync_copy(dsh.at[pl.ds(rbase, RPT)], dvec)
        pltpu.sync_copy(zvec, out_hbm.at[pl.ds(rbase, RPT)])
        pltpu.sync_copy(dvec, out_hbm.at[pl.ds(2 * N_ACC + rbase, RPT)])

    @pl.when(c == 1)
    def _():
        pltpu.sync_copy(osh.at[pl.ds(rbase, RPT)], zvec)
        pltpu.sync_copy(dsh.at[pl.ds(rbase, RPT)], dvec)
        pltpu.sync_copy(zvec, out_hbm.at[pl.ds(N_ACC + rbase, RPT)])
        pltpu.sync_copy(dvec, out_hbm.at[pl.ds(3 * N_ACC + rbase, RPT)])


_sc_layer3 = pl.kernel(
    _sc3_body,
    out_type=jax.ShapeDtypeStruct((4 * N_ACC,), jnp.float32),
    mesh=plsc.VectorSubcoreMesh(**_SC_MESH),
    compiler_params=_SC_PARAMS,
    scratch_types=[
        pltpu.VMEM((SD_PAD,), jnp.float32),
        pltpu.VMEM((N,), jnp.float32),
        pltpu.VMEM((CPW, C), jnp.int32),
        pltpu.VMEM((CPW, C), jnp.int32),
        pltpu.VMEM((1, C), jnp.float32),
        pltpu.VMEM((RPT,), jnp.float32),
        pltpu.VMEM((RPT,), jnp.float32),
        pltpu.VMEM_SHARED((N_ACC,), jnp.float32),
        pltpu.VMEM_SHARED((N_ACC,), jnp.float32),
    ],
)


def _tcf_body(parts_ref, b_ref, out_ref):
    p = parts_ref[...]
    num = p[:, 0:1] + p[:, 1:2]
    den = p[:, 2:3] + p[:, 3:4] + 1e-16
    out_ref[...] = num / den + b_ref[...]


def _tc_final(parts, b3):
    return pl.pallas_call(
        _tcf_body,
        grid=(_GRID,),
        in_specs=[
            pl.BlockSpec((_BLK, 4), lambda i: (i, 0)),
            pl.BlockSpec((1, 1), lambda i: (0, 0)),
        ],
        out_specs=pl.BlockSpec((_BLK, 1), lambda i: (i, 0)),
        out_shape=jax.ShapeDtypeStruct((N, 1), jnp.float32),
    )(parts, b3)


def _flatten_sd(sd):
    flat = sd.reshape(-1)
    return jnp.concatenate([flat, jnp.zeros((SD_PAD - 2 * N,), jnp.float32)])


def kernel(x, edge_index, W1, a1s, a1d, b1, W2, a2s, a2d, b2, W3, a3s, a3d, b3):
    src = edge_index[0]
    dst = edge_index[1]

    pad = E_PAD - E
    apad = jnp.arange(pad, dtype=jnp.int32)
    src2d = jnp.concatenate([src, (apad * 131) % N]).reshape(NCHUNK, C)
    dst_p = jnp.concatenate([dst, N + apad % (N_ACC - N)])
    dst2d = dst_p.reshape(NCHUNK, C)

    A1 = jnp.stack([a1s, a1d], axis=1)
    A2 = jnp.stack([a2s, a2d], axis=1)
    A3 = jnp.stack([a3s, a3d], axis=1)

    hab, sd1 = _tc_proj1(x, W1, A1)
    src3 = jnp.stack([src2d, src2d + N])
    ee1, d10, d11 = _sc_pha(src2d, dst2d, _flatten_sd(sd1))
    oa1, ob1 = _sc_phb1(src3, dst2d, ee1, d10, d11, hab.reshape(2 * N, 128))

    t2, sd2 = _tc_proj2(oa1[:N], ob1[:N], b1[None, :], W2, A2)
    ee2, d20, d21 = _sc_pha(src2d, dst2d, _flatten_sd(sd2))
    p0, p1 = _sc_phb2(src2d, dst2d, ee2, t2)

    h3, sd3 = _tc_proj3(p0[:N, :64], p1[:N, :64],
                        d20[:N].reshape(N, 1), d21[:N].reshape(N, 1),
                        b2[None, :], W3, A3)
    parts3 = _sc_layer3(src2d, dst2d, _flatten_sd(sd3), h3.reshape(-1))
    parts3 = parts3.reshape(4, N_ACC)[:, :N].T

    return _tc_final(parts3, b3.reshape(1, 1))


# speedup vs baseline: 1.9631x; 1.9629x over previous
import jax
import jax.numpy as jnp
from jax import lax
from jax.experimental import pallas as pl
from jax.experimental.pallas import tpu as pltpu
from jax.experimental.pallas import tpu_sc as plsc

N = 10000
E = 320000
D = 128

NC = 2
NS = 16
L = 16

C = 128
NCHUNK = 2560
E_PAD = NCHUNK * C
CPT = NCHUNK // NS
CPW = NCHUNK // (NC * NS)
N_ACC = 10240
SD_PAD = 2 * N_ACC
RPT = N_ACC // NS
G8 = 8

_SC_PARAMS = pltpu.CompilerParams(needs_layout_passes=False)
_SC_MESH = dict(core_axis_name="c", subcore_axis_name="s")


_BLK = 400
_GRID = N // _BLK


def _tc1_body(x_ref, w_ref, a_ref, hab_ref, sd_ref):
    h = jnp.dot(x_ref[...], w_ref[...], preferred_element_type=jnp.float32)
    hab_ref[...] = jnp.stack([h[:, :128], h[:, 128:]], axis=0)
    sd_ref[...] = jnp.dot(h, a_ref[...], preferred_element_type=jnp.float32)


def _tc_proj1(x, W1, A1):
    return pl.pallas_call(
        _tc1_body,
        grid=(_GRID,),
        in_specs=[
            pl.BlockSpec((_BLK, 128), lambda i: (i, 0)),
            pl.BlockSpec((128, 256), lambda i: (0, 0)),
            pl.BlockSpec((256, 2), lambda i: (0, 0)),
        ],
        out_specs=[
            pl.BlockSpec((2, _BLK, 128), lambda i: (0, i, 0)),
            pl.BlockSpec((_BLK, 2), lambda i: (i, 0)),
        ],
        out_shape=[
            jax.ShapeDtypeStruct((2, N, 128), jnp.float32),
            jax.ShapeDtypeStruct((N, 2), jnp.float32),
        ],
    )(x, W1, A1)


def _tc2_body(oa_ref, ob_ref, b_ref, w_ref, a_ref, t2_ref, sd_ref):
    hcat = jnp.concatenate([oa_ref[...], ob_ref[...]], axis=1) + b_ref[...]
    z = jnp.where(hcat > 0, hcat, jnp.exp(hcat) - 1.0)
    h2 = jnp.dot(z, w_ref[...], preferred_element_type=jnp.float32)
    t2_ref[...] = jnp.concatenate(
        [h2, jnp.zeros((_BLK, 64), jnp.float32)], axis=1)
    sd_ref[...] = jnp.dot(h2, a_ref[...], preferred_element_type=jnp.float32)


def _tc_proj2(oa, ob, b1, W2, A2):
    return pl.pallas_call(
        _tc2_body,
        grid=(_GRID,),
        in_specs=[
            pl.BlockSpec((_BLK, 128), lambda i: (i, 0)),
            pl.BlockSpec((_BLK, 128), lambda i: (i, 0)),
            pl.BlockSpec((1, 256), lambda i: (0, 0)),
            pl.BlockSpec((256, 64), lambda i: (0, 0)),
            pl.BlockSpec((64, 2), lambda i: (0, 0)),
        ],
        out_specs=[
            pl.BlockSpec((_BLK, 128), lambda i: (i, 0)),
            pl.BlockSpec((_BLK, 2), lambda i: (i, 0)),
        ],
        out_shape=[
            jax.ShapeDtypeStruct((N, 128), jnp.float32),
            jax.ShapeDtypeStruct((N, 2), jnp.float32),
        ],
    )(oa, ob, b1, W2, A2)


def _tc3_body(p0_ref, p1_ref, d0_ref, d1_ref, b_ref, w_ref, a_ref,
              h3_ref, sd_ref):
    den = d0_ref[...] + d1_ref[...] + 1e-16
    h2 = (p0_ref[...] + p1_ref[...]) / den
    hcat = h2 + b_ref[...]
    z = jnp.where(hcat > 0, hcat, jnp.exp(hcat) - 1.0)
    h3 = jnp.dot(z, w_ref[...], preferred_element_type=jnp.float32)
    h3_ref[...] = h3
    sd_ref[...] = jnp.dot(h3, a_ref[...], preferred_element_type=jnp.float32)


def _tc_proj3(p0, p1, d0, d1, b2, W3, A3):
    return pl.pallas_call(
        _tc3_body,
        grid=(_GRID,),
        in_specs=[
            pl.BlockSpec((_BLK, 64), lambda i: (i, 0)),
            pl.BlockSpec((_BLK, 64), lambda i: (i, 0)),
            pl.BlockSpec((_BLK, 1), lambda i: (i, 0)),
            pl.BlockSpec((_BLK, 1), lambda i: (i, 0)),
            pl.BlockSpec((1, 64), lambda i: (0, 0)),
            pl.BlockSpec((64, 1), lambda i: (0, 0)),
            pl.BlockSpec((1, 2), lambda i: (0, 0)),
        ],
        out_specs=[
            pl.BlockSpec((_BLK, 1), lambda i: (i, 0)),
            pl.BlockSpec((_BLK, 2), lambda i: (i, 0)),
        ],
        out_shape=[
            jax.ShapeDtypeStruct((N, 1), jnp.float32),
            jax.ShapeDtypeStruct((N, 2), jnp.float32),
        ],
    )(p0, p1, d0, d1, b2, W3, A3)


def _zero_vec(buf, n):
    def zv(k, _):
        buf[pl.ds(k * L, L)] = jnp.zeros((L,), jnp.float32)
        return 0

    lax.fori_loop(0, n // L, zv, 0)


def _zero_rows(rowb, nrows, ncols):
    def zrow(r, _):
        for f in range(ncols // L):
            rowb[r, pl.ds(f * L, L)] = jnp.zeros((L,), jnp.float32)
        return 0

    lax.fori_loop(0, nrows, zrow, 0)


def _pha_body(src2d, dst2d, sd_hbm, ee2d, d0_hbm, d1_hbm,
              sdv, srcb, dstb, eec, zvec, dsh):
    c = lax.axis_index("c")
    s = lax.axis_index("s")
    cbase = (c * NS + s) * CPW

    pltpu.sync_copy(sd_hbm, sdv)
    pltpu.sync_copy(src2d.at[pl.ds(cbase, CPW)], srcb)
    pltpu.sync_copy(dst2d.at[pl.ds(cbase, CPW)], dstb)
    _zero_vec(zvec, RPT)
    pltpu.sync_copy(zvec, dsh.at[pl.ds(s * RPT, RPT)])
    plsc.subcore_barrier()

    def group(ii, _):
        def chunk(slot, _):
            i = ii * G8 + slot

            def vec(j, _):
                off = j * L
                vsrc = srcb[i, pl.ds(off, L)]
                vdst = dstb[i, pl.ds(off, L)]
                ss = plsc.load_gather(sdv, [vsrc * 2])
                dd = plsc.load_gather(sdv, [vdst * 2 + 1])
                e = ss + dd
                e = jnp.maximum(e, e * 0.2)
                eec[slot, pl.ds(off, L)] = jnp.exp(e)
                return 0

            lax.fori_loop(0, C // L, vec, 0)
            pltpu.sync_copy(eec.at[slot], dsh.at[dstb.at[i]], add=True)
            return 0

        lax.fori_loop(0, G8, chunk, 0)
        pltpu.sync_copy(eec, ee2d.at[pl.ds(cbase + ii * G8, G8)])
        return 0

    lax.fori_loop(0, CPW // G8, group, 0)
    plsc.subcore_barrier()

    @pl.when(c == 0)
    def _():
        pltpu.sync_copy(dsh.at[pl.ds(s * RPT, RPT)],
                        d0_hbm.at[pl.ds(s * RPT, RPT)])

    @pl.when(c == 1)
    def _():
        pltpu.sync_copy(dsh.at[pl.ds(s * RPT, RPT)],
                        d1_hbm.at[pl.ds(s * RPT, RPT)])


_sc_pha = pl.kernel(
    _pha_body,
    out_type=[
        jax.ShapeDtypeStruct((NCHUNK, C), jnp.float32),
        jax.ShapeDtypeStruct((N_ACC,), jnp.float32),
        jax.ShapeDtypeStruct((N_ACC,), jnp.float32),
    ],
    mesh=plsc.VectorSubcoreMesh(**_SC_MESH),
    compiler_params=_SC_PARAMS,
    scratch_types=[
        pltpu.VMEM((SD_PAD,), jnp.float32),
        pltpu.VMEM((CPW, C), jnp.int32),
        pltpu.VMEM((CPW, C), jnp.int32),
        pltpu.VMEM((G8, C), jnp.float32),
        pltpu.VMEM((RPT,), jnp.float32),
        pltpu.VMEM_SHARED((N_ACC,), jnp.float32),
    ],
)


def _agg_chunks(src_sl, dst2d, ee2d, tab_hbm, cbase, nchunks,
                idx3, dst3, ee3, rowb2, acc, gsem0, gsem1, psem,
                nscale=128 // L, goff=0):
    ngroups = nchunks // G8

    def row_of(ii):
        return cbase + lax.rem(ii + goff, ngroups) * G8

    def meta_issue(ii):
        row0 = row_of(ii)
        b = lax.rem(ii, 2)
        pltpu.async_copy(src_sl(row0), idx3.at[b], psem)
        pltpu.async_copy(dst2d.at[pl.ds(row0, G8)], dst3.at[b], psem)
        pltpu.async_copy(ee2d.at[pl.ds(row0, G8)], ee3.at[b], psem)

    def meta_drain(ii):
        row0 = row_of(ii)
        b = lax.rem(ii, 2)
        pltpu.make_async_copy(src_sl(row0), idx3.at[b], psem).wait()
        pltpu.make_async_copy(dst2d.at[pl.ds(row0, G8)], dst3.at[b],
                              psem).wait()
        pltpu.make_async_copy(ee2d.at[pl.ds(row0, G8)], ee3.at[b],
                              psem).wait()

    def gather_issue(g, sem):
        ii = g // G8
        slot = lax.rem(g, G8)
        gb = lax.rem(g, 2)
        pltpu.async_copy(tab_hbm.at[idx3.at[lax.rem(ii, 2), slot]],
                         rowb2.at[gb], sem)

    def gather_wait(g, sem):
        gb = lax.rem(g, 2)
        pltpu.make_async_copy(tab_hbm.at[idx3.at[0, 0]],
                              rowb2.at[gb], sem).wait()

    meta_issue(0)
    meta_drain(0)
    gather_issue(0, gsem0)
    gather_issue(1, gsem1)

    def group(ii, _):
        @pl.when(ii + 1 < ngroups)
        def _():
            meta_issue(ii + 1)

        ib = lax.rem(ii, 2)

        def chunk(slot, _):
            g = ii * G8 + slot
            gb = lax.rem(g, 2)

            @pl.when(gb == 0)
            def _():
                gather_wait(g, gsem0)

            @pl.when(gb == 1)
            def _():
                gather_wait(g, gsem1)

            @plsc.parallel_loop(0, C // L, step=1, unroll=2)
            def _(grp):
                av = ee3[ib, slot, pl.ds(grp * L, L)]
                for r in range(L):
                    a = av[r]
                    row = grp * L + r
                    vals = [rowb2[gb, row, pl.ds(f * L, L)]
                            for f in range(nscale)]
                    for f in range(nscale):
                        rowb2[gb, row, pl.ds(f * L, L)] = vals[f] * a

            pltpu.sync_copy(rowb2.at[gb], acc.at[dst3.at[ib, slot]],
                            add=True)

            @pl.when(slot == G8 - 3)
            def _():
                @pl.when(ii + 1 < ngroups)
                def _():
                    meta_drain(ii + 1)

            @pl.when(g + 2 < nchunks)
            def _():
                @pl.when(gb == 0)
                def _():
                    gather_issue(g + 2, gsem0)

                @pl.when(gb == 1)
                def _():
                    gather_issue(g + 2, gsem1)

            return 0

        lax.fori_loop(0, G8, chunk, 0)
        return 0

    lax.fori_loop(0, ngroups, group, 0)


def _phb1_body(src3, dst2d, ee2d, d0_hbm, d1_hbm, hab, oa, ob,
               idx3, dst3, ee3, rowb2, dvec, dvec2, acc,
               gsem0, gsem1, psem):
    c = lax.axis_index("c")
    s = lax.axis_index("s")
    rbase = s * RPT

    _zero_rows(rowb2.at[0], C, 128)
    for k in range(RPT // C):
        pltpu.sync_copy(rowb2.at[0], acc.at[pl.ds(rbase + k * C, C)])
    plsc.subcore_barrier()

    _agg_chunks(lambda row0: src3.at[c, pl.ds(row0, G8)],
                dst2d, ee2d, hab, s * CPT, CPT,
                idx3, dst3, ee3, rowb2, acc, gsem0, gsem1, psem,
                goff=c * ((CPT // G8) // 2))

    plsc.subcore_barrier()

    def wchunk(k, _):
        r0 = rbase + k * C
        pltpu.sync_copy(acc.at[pl.ds(r0, C)], rowb2.at[0])
        pltpu.sync_copy(d0_hbm.at[pl.ds(r0, C)], dvec)
        pltpu.sync_copy(d1_hbm.at[pl.ds(r0, C)], dvec2)

        def sgroup(g, _):
            off = g * L
            rec = 1.0 / (dvec[pl.ds(off, L)] + dvec2[pl.ds(off, L)] + 1e-16)
            for r in range(L):
                a = rec[r]
                row = off + r
                for f in range(128 // L):
                    rowb2[0, row, pl.ds(f * L, L)] = (
                        rowb2[0, row, pl.ds(f * L, L)] * a)
            return 0

        lax.fori_loop(0, C // L, sgroup, 0)

        @pl.when(c == 0)
        def _():
            pltpu.sync_copy(rowb2.at[0], oa.at[pl.ds(r0, C)])

        @pl.when(c == 1)
        def _():
            pltpu.sync_copy(rowb2.at[0], ob.at[pl.ds(r0, C)])

        return 0

    lax.fori_loop(0, RPT // C, wchunk, 0)


_sc_phb1 = pl.kernel(
    _phb1_body,
    out_type=[
        jax.ShapeDtypeStruct((N_ACC, 128), jnp.float32),
        jax.ShapeDtypeStruct((N_ACC, 128), jnp.float32),
    ],
    mesh=plsc.VectorSubcoreMesh(**_SC_MESH),
    compiler_params=_SC_PARAMS,
    scratch_types=[
        pltpu.VMEM((2, G8, C), jnp.int32),
        pltpu.VMEM((2, G8, C), jnp.int32),
        pltpu.VMEM((2, G8, C), jnp.float32),
        pltpu.VMEM((2, C, 128), jnp.float32),
        pltpu.VMEM((C,), jnp.float32),
        pltpu.VMEM((C,), jnp.float32),
        pltpu.VMEM_SHARED((N_ACC, 128), jnp.float32),
        pltpu.SemaphoreType.DMA,
        pltpu.SemaphoreType.DMA,
        pltpu.SemaphoreType.DMA,
    ],
)


def _phb2_body(src2d, dst2d, ee2d, t2, p0, p1,
               idx3, dst3, ee3, rowb2, acc, gsem0, gsem1, psem):
    c = lax.axis_index("c")
    s = lax.axis_index("s")
    rbase = s * RPT
    cbase = (c * NS + s) * CPW

    _zero_rows(rowb2.at[0], C, 128)
    for k in range(RPT // C):
        pltpu.sync_copy(rowb2.at[0], acc.at[pl.ds(rbase + k * C, C)])
    plsc.subcore_barrier()

    _agg_chunks(lambda row0: src2d.at[pl.ds(row0, G8)],
                dst2d, ee2d, t2, cbase, CPW,
                idx3, dst3, ee3, rowb2, acc, gsem0, gsem1, psem,
                nscale=64 // L)
    plsc.subcore_barrier()

    def wchunk(k, _):
        r0 = rbase + k * C
        pltpu.sync_copy(acc.at[pl.ds(r0, C)], rowb2.at[0])

        @pl.when(c == 0)
        def _():
            pltpu.sync_copy(rowb2.at[0], p0.at[pl.ds(r0, C)])

        @pl.when(c == 1)
        def _():
            pltpu.sync_copy(rowb2.at[0], p1.at[pl.ds(r0, C)])

        return 0

    lax.fori_loop(0, RPT // C, wchunk, 0)


_sc_phb2 = pl.kernel(
    _phb2_body,
    out_type=[
        jax.ShapeDtypeStruct((N_ACC, 128), jnp.float32),
        jax.ShapeDtypeStruct((N_ACC, 128), jnp.float32),
    ],
    mesh=plsc.VectorSubcoreMesh(**_SC_MESH),
    compiler_params=_SC_PARAMS,
    scratch_types=[
        pltpu.VMEM((2, G8, C), jnp.int32),
        pltpu.VMEM((2, G8, C), jnp.int32),
        pltpu.VMEM((2, G8, C), jnp.float32),
        pltpu.VMEM((2, C, 128), jnp.float32),
        pltpu.VMEM_SHARED((N_ACC, 128), jnp.float32),
        pltpu.SemaphoreType.DMA,
        pltpu.SemaphoreType.DMA,
        pltpu.SemaphoreType.DMA,
    ],
)


def _sc3_body(src2d, dst2d, sd_hbm, h3_hbm, out_hbm,
              sdv, h3v, srcb, dstb, eec, zvec, dvec, osh, dsh):
    c = lax.axis_index("c")
    s = lax.axis_index("s")
    cbase = (c * NS + s) * CPW
    rbase = s * RPT

    pltpu.sync_copy(sd_hbm, sdv)
    pltpu.sync_copy(h3_hbm, h3v)
    pltpu.sync_copy(src2d.at[pl.ds(cbase, CPW)], srcb)
    pltpu.sync_copy(dst2d.at[pl.ds(cbase, CPW)], dstb)
    _zero_vec(zvec, RPT)
    pltpu.sync_copy(zvec, dsh.at[pl.ds(rbase, RPT)])
    pltpu.sync_copy(zvec, osh.at[pl.ds(rbase, RPT)])
    plsc.subcore_barrier()

    def chunk(i, _):
        def vec(j, _):
            off = j * L
            vsrc = srcb[i, pl.ds(off, L)]
            vdst = dstb[i, pl.ds(off, L)]
            ss = plsc.load_gather(sdv, [vsrc * 2])
            dd = plsc.load_gather(sdv, [vdst * 2 + 1])
            e = ss + dd
            e = jnp.maximum(e, e * 0.2)
            eec[0, pl.ds(off, L)] = jnp.exp(e)
            return 0

        lax.fori_loop(0, C // L, vec, 0)
        pltpu.sync_copy(eec.at[0], dsh.at[dstb.at[i]], add=True)

        def vec2(j, _):
            off = j * L
            vsrc = srcb[i, pl.ds(off, L)]
            vh = plsc.load_gather(h3v, [vsrc])
            eec[0, pl.ds(off, L)] = eec[0, pl.ds(off, L)] * vh
            return 0

        lax.fori_loop(0, C // L, vec2, 0)
        pltpu.sync_copy(eec.at[0], osh.at[dstb.at[i]], add=True)
        return 0

    lax.fori_loop(0, CPW, chunk, 0)
    plsc.subcore_barrier()

    @pl.when(c == 0)
    def _():
        pltpu.sync_copy(osh.at[pl.ds(rbase, RPT)], zvec)
        pltpu.sync_copy(dsh.at[pl.ds(rbase, RPT)], dvec)
        pltpu.sync_copy(zvec, out_hbm.at[pl.ds(rbase, RPT)])
        pltpu.sync_copy(dvec, out_hbm.at[pl.ds(2 * N_ACC + rbase, RPT)])

    @pl.when(c == 1)
    def _():
        pltpu.sync_copy(osh.at[pl.ds(rbase, RPT)], zvec)
        pltpu.sync_copy(dsh.at[pl.ds(rbase, RPT)], dvec)
        pltpu.sync_copy(zvec, out_hbm.at[pl.ds(N_ACC + rbase, RPT)])
        pltpu.sync_copy(dvec, out_hbm.at[pl.ds(3 * N_ACC + rbase, RPT)])


_sc_layer3 = pl.kernel(
    _sc3_body,
    out_type=jax.ShapeDtypeStruct((4 * N_ACC,), jnp.float32),
    mesh=plsc.VectorSubcoreMesh(**_SC_MESH),
    compiler_params=_SC_PARAMS,
    scratch_types=[
        pltpu.VMEM((SD_PAD,), jnp.float32),
        pltpu.VMEM((N,), jnp.float32),
        pltpu.VMEM((CPW, C), jnp.int32),
        pltpu.VMEM((CPW, C), jnp.int32),
        pltpu.VMEM((1, C), jnp.float32),
        pltpu.VMEM((RPT,), jnp.float32),
        pltpu.VMEM((RPT,), jnp.float32),
        pltpu.VMEM_SHARED((N_ACC,), jnp.float32),
        pltpu.VMEM_SHARED((N_ACC,), jnp.float32),
    ],
)


def _tcf_body(parts_ref, b_ref, out_ref):
    p = parts_ref[...]
    num = p[:, 0:1] + p[:, 1:2]
    den = p[:, 2:3] + p[:, 3:4] + 1e-16
    out_ref[...] = num / den + b_ref[...]


def _tc_final(parts, b3):
    return pl.pallas_call(
        _tcf_body,
        grid=(_GRID,),
        in_specs=[
            pl.BlockSpec((_BLK, 4), lambda i: (i, 0)),
            pl.BlockSpec((1, 1), lambda i: (0, 0)),
        ],
        out_specs=pl.BlockSpec((_BLK, 1), lambda i: (i, 0)),
        out_shape=jax.ShapeDtypeStruct((N, 1), jnp.float32),
    )(parts, b3)


def _flatten_sd(sd):
    flat = sd.reshape(-1)
    return jnp.concatenate([flat, jnp.zeros((SD_PAD - 2 * N,), jnp.float32)])


def kernel(x, edge_index, W1, a1s, a1d, b1, W2, a2s, a2d, b2, W3, a3s, a3d, b3):
    src = edge_index[0]
    dst = edge_index[1]

    pad = E_PAD - E
    apad = jnp.arange(pad, dtype=jnp.int32)
    src2d = jnp.concatenate([src, (apad * 131) % N]).reshape(NCHUNK, C)
    dst_p = jnp.concatenate([dst, N + apad % (N_ACC - N)])
    dst2d = dst_p.reshape(NCHUNK, C)

    A1 = jnp.stack([a1s, a1d], axis=1)
    A2 = jnp.stack([a2s, a2d], axis=1)
    A3 = jnp.stack([a3s, a3d], axis=1)

    hab, sd1 = _tc_proj1(x, W1, A1)
    src3 = jnp.stack([src2d, src2d + N])
    ee1, d10, d11 = _sc_pha(src2d, dst2d, _flatten_sd(sd1))
    oa1, ob1 = _sc_phb1(src3, dst2d, ee1, d10, d11, hab.reshape(2 * N, 128))

    t2, sd2 = _tc_proj2(oa1[:N], ob1[:N], b1[None, :], W2, A2)
    ee2, d20, d21 = _sc_pha(src2d, dst2d, _flatten_sd(sd2))
    p0, p1 = _sc_phb2(src2d, dst2d, ee2, t2)

    h3, sd3 = _tc_proj3(p0[:N, :64], p1[:N, :64],
                        d20[:N].reshape(N, 1), d21[:N].reshape(N, 1),
                        b2[None, :], W3, A3)
    parts3 = _sc_layer3(src2d, dst2d, _flatten_sd(sd3), h3.reshape(-1))
    parts3 = parts3.reshape(4, N_ACC)[:, :N].T

    return _tc_final(parts3, b3.reshape(1, 1))
